# Initial kernel scaffold; baseline (speedup 1.0000x reference)
import jax
import jax.numpy as jnp
from jax.experimental import pallas as pl


def kernel(idx, table):
    raise NotImplementedError("write your pallas kernel here")


# speedup vs baseline: 1.4137x; 1.4137x over previous
import functools

import jax
import jax.numpy as jnp
from jax import lax
from jax.experimental import pallas as pl
from jax.experimental.pallas import tpu as pltpu
from jax.experimental.pallas import tpu_sc as plsc

VOCAB = 1000
D = 1000
BT = 4096 * 20
NW = 32
B_PER_W = BT // NW
CHUNK = 64
NCHUNK = B_PER_W // CHUNK

_mesh = plsc.VectorSubcoreMesh(core_axis_name="c", subcore_axis_name="s")


@functools.partial(
    pl.kernel,
    mesh=_mesh,
    out_type=jax.ShapeDtypeStruct((BT, D), jnp.float32),
    compiler_params=pltpu.CompilerParams(use_tc_tiling_on_sc=False),
    scratch_types=[
        pltpu.VMEM((B_PER_W,), jnp.int32),
        pltpu.VMEM((CHUNK, D), jnp.float32),
        pltpu.VMEM((CHUNK, D), jnp.float32),
        pltpu.SemaphoreType.DMA,
        pltpu.SemaphoreType.DMA,
        pltpu.SemaphoreType.DMA,
        pltpu.SemaphoreType.DMA,
    ],
)
def _gather_kernel(idx_hbm, table_hbm, out_hbm, idx_v, buf0, buf1, g0, g1, w0, w1):
    wid = lax.axis_index("s") * 2 + lax.axis_index("c")
    base = wid * B_PER_W
    pltpu.sync_copy(idx_hbm.at[pl.ds(base, B_PER_W)], idx_v)

    bufs = (buf0, buf1)
    gsems = (g0, g1)
    wsems = (w0, w1)

    def gather(c, b):
        pltpu.async_copy(
            table_hbm.at[idx_v.at[pl.ds(c * CHUNK, CHUNK)]], bufs[b], gsems[b])

    def wait_gather(b):
        pltpu.make_async_copy(
            table_hbm.at[pl.ds(0, CHUNK)], bufs[b], gsems[b]).wait()

    def write(c, b):
        pltpu.async_copy(
            bufs[b], out_hbm.at[pl.ds(base + c * CHUNK, CHUNK)], wsems[b])

    def wait_write(b):
        pltpu.make_async_copy(
            bufs[b], out_hbm.at[pl.ds(base, CHUNK)], wsems[b]).wait()

    gather(0, 0)
    gather(1, 1)

    def body(i, carry):
        c = 2 * i
        wait_gather(0)
        write(c, 0)
        wait_gather(1)
        write(c + 1, 1)
        wait_write(0)

        @pl.when(i + 1 < NCHUNK // 2)
        def _():
            gather(c + 2, 0)

        wait_write(1)

        @pl.when(i + 1 < NCHUNK // 2)
        def _():
            gather(c + 3, 1)

        return carry

    lax.fori_loop(0, NCHUNK // 2, body, 0)


def kernel(idx, table):
    out = _gather_kernel(idx.reshape(BT), table)
    return out.reshape(idx.shape[0], idx.shape[1], D)


# speedup vs baseline: 1.4902x; 1.0541x over previous
import functools

import jax
import jax.numpy as jnp
from jax import lax
from jax.experimental import pallas as pl
from jax.experimental.pallas import tpu as pltpu
from jax.experimental.pallas import tpu_sc as plsc

VOCAB = 1000
D = 1000
B = 4096
T = 20
NW = 32
NVT = D // 8
NBLK = T * NVT
BLK = 32 * 8 * 128

_mesh = plsc.VectorSubcoreMesh(core_axis_name="c", subcore_axis_name="s")


@functools.partial(
    pl.kernel,
    mesh=_mesh,
    out_type=jax.ShapeDtypeStruct((NBLK, BLK), jnp.float32),
    compiler_params=pltpu.CompilerParams(
        use_tc_tiling_on_sc=False, needs_layout_passes=False),
    scratch_types=[
        pltpu.VMEM((8, D), jnp.float32),
        pltpu.VMEM((8, D), jnp.float32),
        pltpu.VMEM((B,), jnp.int32),
        pltpu.VMEM((B,), jnp.int32),
        pltpu.VMEM((BLK,), jnp.float32),
        pltpu.VMEM((BLK,), jnp.float32),
        pltpu.SemaphoreType.DMA,
        pltpu.SemaphoreType.DMA,
        pltpu.SemaphoreType.DMA,
        pltpu.SemaphoreType.DMA,
    ],
)
def _gather_kernel(idxT_hbm, tableT_hbm, out_hbm,
                   rbA, rbB, ibA, ibB, bbA, bbB, lA, lB, wA, wB):
    w = lax.axis_index("s") * 2 + lax.axis_index("c")

    rbs = (rbA, rbB)
    ibs = (ibA, ibB)
    bbs = (bbA, bbB)
    lsems = (lA, lB)
    wsems = (wA, wB)

    def load(nb, s):
        t = nb // NVT
        vt = nb - t * NVT
        pltpu.async_copy(tableT_hbm.at[pl.ds(vt * 8, 8)], rbs[s], lsems[s])
        pltpu.async_copy(idxT_hbm.at[t], ibs[s], lsems[s])

    def wait_load(s):
        pltpu.make_async_copy(tableT_hbm.at[pl.ds(0, 8)], rbs[s], lsems[s]).wait()
        pltpu.make_async_copy(idxT_hbm.at[0], ibs[s], lsems[s]).wait()

    def write(nb, s):
        pltpu.async_copy(bbs[s], out_hbm.at[nb], wsems[s])

    def wait_write(s):
        pltpu.make_async_copy(bbs[s], out_hbm.at[0], wsems[s]).wait()

    def compute(s):
        rb, ib, bb = rbs[s], ibs[s], bbs[s]

        def body_bt(bt, carry):
            base = bt * 1024
            for j in range(8):
                idx16 = ib[pl.ds(bt * 128 + j * 16, 16)]
                for vi in range(8):
                    val = plsc.load_gather(rb.at[vi], [idx16])
                    bb[pl.ds(base + vi * 128 + j * 16, 16)] = val
            return carry

        lax.fori_loop(0, 32, body_bt, 0)

    load(w, 0)

    @pl.when(w + 32 < NBLK)
    def _():
        load(w + 32, 1)

    def body(k2, carry):
        for s in range(2):
            nb = w + 32 * (2 * k2 + s)

            @pl.when(nb < NBLK)
            def _():
                wait_load(s)

                @pl.when(k2 > 0)
                def _():
                    wait_write(s)

                compute(s)
                write(nb, s)

                @pl.when(nb + 64 < NBLK)
                def _():
                    load(nb + 64, s)

        return carry

    lax.fori_loop(0, (NBLK // NW + 2) // 2, body, 0)
    wait_write(0)
    wait_write(1)


def kernel(idx, table):
    idxT = idx.T
    tableT = table.T
    out2 = _gather_kernel(idxT, tableT)
    out5 = out2.reshape(T, NVT, 32, 8, 128)
    return out5.transpose(2, 4, 0, 1, 3).reshape(B, T, D)


# speedup vs baseline: 4.1313x; 2.7723x over previous
import functools

import jax
import jax.numpy as jnp
from jax import lax
from jax.experimental import pallas as pl
from jax.experimental.pallas import tpu as pltpu
from jax.experimental.pallas import tpu_sc as plsc

VOCAB = 1000
D = 1000
B = 4096
T = 20
NW = 32
NVT = D // 8
NBLK = T * NVT
BLK = 32 * 8 * 128

_mesh = plsc.VectorSubcoreMesh(core_axis_name="c", subcore_axis_name="s")


@functools.partial(
    pl.kernel,
    mesh=_mesh,
    out_type=jax.ShapeDtypeStruct((NBLK, BLK), jnp.float32),
    compiler_params=pltpu.CompilerParams(
        use_tc_tiling_on_sc=False, needs_layout_passes=False),
    scratch_types=[
        pltpu.VMEM((8, D), jnp.float32),
        pltpu.VMEM((8, D), jnp.float32),
        pltpu.VMEM((B,), jnp.int32),
        pltpu.VMEM((B,), jnp.int32),
        pltpu.VMEM((BLK,), jnp.float32),
        pltpu.VMEM((BLK,), jnp.float32),
        pltpu.SemaphoreType.DMA,
        pltpu.SemaphoreType.DMA,
        pltpu.SemaphoreType.DMA,
        pltpu.SemaphoreType.DMA,
    ],
)
def _gather_kernel(idxT_hbm, tableT_hbm, out_hbm,
                   rbA, rbB, ibA, ibB, bbA, bbB, lA, lB, wA, wB):
    w = lax.axis_index("s") * 2 + lax.axis_index("c")

    rbs = (rbA, rbB)
    ibs = (ibA, ibB)
    bbs = (bbA, bbB)
    lsems = (lA, lB)
    wsems = (wA, wB)

    def load(nb, s):
        t = nb // NVT
        vt = nb - t * NVT
        pltpu.async_copy(tableT_hbm.at[pl.ds(vt * 8, 8)], rbs[s], lsems[s])
        pltpu.async_copy(idxT_hbm.at[t], ibs[s], lsems[s])

    def wait_load(s):
        pltpu.make_async_copy(tableT_hbm.at[pl.ds(0, 8)], rbs[s], lsems[s]).wait()
        pltpu.make_async_copy(idxT_hbm.at[0], ibs[s], lsems[s]).wait()

    def write(nb, s):
        pltpu.async_copy(bbs[s], out_hbm.at[nb], wsems[s])

    def wait_write(s):
        pltpu.make_async_copy(bbs[s], out_hbm.at[0], wsems[s]).wait()

    def compute(s):
        rb, ib, bb = rbs[s], ibs[s], bbs[s]

        @plsc.parallel_loop(0, 32, 1, unroll=2)
        def body_bt(bt):
            base = bt * 1024
            for j in range(8):
                idx16 = ib[pl.ds(bt * 128 + j * 16, 16)]
                vals = [plsc.load_gather(rb.at[vi], [idx16]) for vi in range(8)]
                for vi in range(8):
                    bb[pl.ds(base + vi * 128 + j * 16, 16)] = vals[vi]

    load(w, 0)

    @pl.when(w + 32 < NBLK)
    def _():
        load(w + 32, 1)

    def body(k2, carry):
        for s in range(2):
            nb = w + 32 * (2 * k2 + s)

            @pl.when(nb < NBLK)
            def _():
                wait_load(s)

                @pl.when(k2 > 0)
                def _():
                    wait_write(s)

                compute(s)
                write(nb, s)

                @pl.when(nb + 64 < NBLK)
                def _():
                    load(nb + 64, s)

        return carry

    lax.fori_loop(0, (NBLK // NW + 2) // 2, body, 0)
    wait_write(0)
    wait_write(1)


def kernel(idx, table):
    idxT = idx.T
    tableT = table.T
    out2 = _gather_kernel(idxT, tableT)
    out5 = out2.reshape(T, NVT, 32, 8, 128)
    return out5.transpose(2, 4, 0, 1, 3).reshape(B, T, D)
